# baseline (speedup 1.0000x reference)
import functools

import jax
import jax.numpy as jnp
from jax import lax
from jax.experimental import pallas as pl
from jax.experimental.pallas import tpu as pltpu
from jax.experimental.pallas import tpu_sc as plsc

EPS = 1e-12
NC, NS, L = 2, 16, 16
NW = NC * NS
RB = 32


def _rsqrt(v):
    i = plsc.bitcast(v, jnp.int32)
    i = jnp.int32(0x5F3759DF) - (i >> 1)
    y = plsc.bitcast(i, jnp.float32)
    for _ in range(3):
        y = y * (1.5 - 0.5 * v * y * y)
    return y


def _sc_kernel(s, h, nblk, pos_hbm, type_hbm, w_hbm, b_hbm, out_hbm,
               row_v, t_v, w_v, b_v, red_v):
    nch = h // L
    wid = lax.axis_index("s") * NC + lax.axis_index("c")
    rows_per_w = s // NW
    base = wid * rows_per_w

    pltpu.sync_copy(type_hbm.at[0], t_v)
    pltpu.sync_copy(w_hbm, w_v)
    pltpu.sync_copy(b_hbm, b_v)

    def do_block(blk, _):
        r0 = base + blk * RB
        pltpu.sync_copy(pos_hbm.at[pl.ds(r0, RB)], row_v)

        def do_row(r, _):
            acc = jnp.zeros((L,), jnp.float32)
            acc2 = jnp.zeros((L,), jnp.float32)
            for j in range(nch):
                x = row_v[r, pl.ds(j * L, L)] + t_v[pl.ds(j * L, L)]
                row_v[r, pl.ds(j * L, L)] = x
                acc = acc + x
                acc2 = acc2 + x * x
            s1 = plsc.cumsum(acc)[L - 1]
            s2 = plsc.cumsum(acc2)[L - 1]
            mean = s1 * (1.0 / h)
            var = s2 * (1.0 / h) - mean * mean
            rstd = _rsqrt(jnp.broadcast_to(var + EPS, (L,)))
            mvec = jnp.broadcast_to(mean, (L,))
            for j in range(nch):
                x = row_v[r, pl.ds(j * L, L)]
                y = (x - mvec) * rstd
                y = y * w_v[pl.ds(j * L, L)] + b_v[pl.ds(j * L, L)]
                row_v[r, pl.ds(j * L, L)] = y
            return 0

        lax.fori_loop(0, RB, do_row, 0)
        for bb in range(out_hbm.shape[0]):
            pltpu.sync_copy(row_v, out_hbm.at[bb, pl.ds(r0, RB)])
        return 0

    lax.fori_loop(0, nblk, do_block, 0)


def kernel(input_ids, pos_table, type_table, ln_weight, ln_bias):
    b, s = input_ids.shape
    h = pos_table.shape[1]
    nblk = (s // NW) // RB
    mesh = plsc.VectorSubcoreMesh(core_axis_name="c", subcore_axis_name="s")
    run = pl.kernel(
        functools.partial(_sc_kernel, s, h, nblk),
        mesh=mesh,
        compiler_params=pltpu.CompilerParams(needs_layout_passes=False),
        out_type=jax.ShapeDtypeStruct((b, s, h), jnp.float32),
        scratch_types=[
            pltpu.VMEM((RB, h), jnp.float32),
            pltpu.VMEM((h,), jnp.float32),
            pltpu.VMEM((h,), jnp.float32),
            pltpu.VMEM((h,), jnp.float32),
            pltpu.VMEM((2 * L,), jnp.float32),
        ],
    )
    return run(pos_table[:s], type_table, ln_weight, ln_bias)


# speedup vs baseline: 1.0654x; 1.0654x over previous
import functools

import jax
import jax.numpy as jnp
from jax import lax
from jax.experimental import pallas as pl
from jax.experimental.pallas import tpu as pltpu
from jax.experimental.pallas import tpu_sc as plsc

EPS = 1e-12
NC, NS, L = 2, 16, 16
NW = NC * NS
RB = 64


def _rsqrt(v):
    i = plsc.bitcast(v, jnp.int32)
    i = jnp.int32(0x5F3759DF) - (i >> 1)
    y = plsc.bitcast(i, jnp.float32)
    for _ in range(3):
        y = y * (1.5 - 0.5 * v * y * y)
    return y


def _sc_kernel(s, h, nblk, nb, pos_hbm, type_hbm, w_hbm, b_hbm, out_hbm,
               row_v, t_v, w_v, b_v, in_sem, out_sem):
    nch = h // L
    wid = lax.axis_index("s") * NC + lax.axis_index("c")
    base = wid * (s // NW)

    pltpu.sync_copy(type_hbm.at[0], t_v)
    pltpu.sync_copy(w_hbm, w_v)
    pltpu.sync_copy(b_hbm, b_v)

    def in_copy(slot, blk):
        return pltpu.make_async_copy(
            pos_hbm.at[pl.ds(base + blk * RB, RB)], row_v.at[slot],
            in_sem.at[slot])

    def out_copy(slot, blk, bb):
        return pltpu.make_async_copy(
            row_v.at[slot], out_hbm.at[bb, pl.ds(base + blk * RB, RB)],
            out_sem.at[slot])

    def do_row(slot, r, _):
        acc = jnp.zeros((L,), jnp.float32)
        acc2 = jnp.zeros((L,), jnp.float32)
        for j in range(nch):
            x = row_v[slot, r, pl.ds(j * L, L)] + t_v[pl.ds(j * L, L)]
            row_v[slot, r, pl.ds(j * L, L)] = x
            acc = acc + x
            acc2 = acc2 + x * x
        s1 = plsc.cumsum(acc)[L - 1]
        s2 = plsc.cumsum(acc2)[L - 1]
        mean = s1 * (1.0 / h)
        var = s2 * (1.0 / h) - mean * mean
        rstd = _rsqrt(jnp.broadcast_to(var + EPS, (L,)))
        mvec = jnp.broadcast_to(mean, (L,))
        for j in range(nch):
            x = row_v[slot, r, pl.ds(j * L, L)]
            y = (x - mvec) * rstd
            y = y * w_v[pl.ds(j * L, L)] + b_v[pl.ds(j * L, L)]
            row_v[slot, r, pl.ds(j * L, L)] = y
        return 0

    in_copy(0, 0).start()
    for blk in range(nblk):
        slot = blk % 2
        if blk + 1 < nblk:
            if blk >= 1:
                for bb in range(nb):
                    out_copy(1 - slot, blk - 1, bb).wait()
            in_copy(1 - slot, blk + 1).start()
        in_copy(slot, blk).wait()
        lax.fori_loop(0, RB, functools.partial(do_row, slot), 0)
        for bb in range(nb):
            out_copy(slot, blk, bb).start()
    for bb in range(nb):
        if nblk >= 2:
            out_copy((nblk - 2) % 2, nblk - 2, bb).wait()
        out_copy((nblk - 1) % 2, nblk - 1, bb).wait()


def kernel(input_ids, pos_table, type_table, ln_weight, ln_bias):
    b, s = input_ids.shape
    h = pos_table.shape[1]
    nblk = (s // NW) // RB
    mesh = plsc.VectorSubcoreMesh(core_axis_name="c", subcore_axis_name="s")
    run = pl.kernel(
        functools.partial(_sc_kernel, s, h, nblk, b),
        mesh=mesh,
        compiler_params=pltpu.CompilerParams(needs_layout_passes=False),
        out_type=jax.ShapeDtypeStruct((b, s, h), jnp.float32),
        scratch_types=[
            pltpu.VMEM((2, RB, h), jnp.float32),
            pltpu.VMEM((h,), jnp.float32),
            pltpu.VMEM((h,), jnp.float32),
            pltpu.VMEM((h,), jnp.float32),
            pltpu.SemaphoreType.DMA((2,)),
            pltpu.SemaphoreType.DMA((2,)),
        ],
    )
    return run(pos_table[:s], type_table, ln_weight, ln_bias)
